# baseline (speedup 1.0000x reference)
import jax
import jax.numpy as jnp
from jax.experimental import pallas as pl
from jax.experimental.pallas import tpu as pltpu

_LANES = 128


def _body(t_ref, alpha_ref, x_ref, n_ref, out_ref):
    b = pl.program_id(0)
    a = alpha_ref[t_ref[b]]
    sa = jnp.sqrt(a)
    sn = jnp.sqrt(1.0 - a)
    out_ref[...] = sa * x_ref[...] + sn * n_ref[...]


def kernel(x_0, noise, t, alpha_cumprod):
    B = x_0.shape[0]
    per = x_0.size // B
    sub = per // _LANES
    x2 = x_0.reshape(B, sub, _LANES)
    n2 = noise.reshape(B, sub, _LANES)
    out = pl.pallas_call(
        _body,
        grid=(B,),
        in_specs=[
            pl.BlockSpec(memory_space=pltpu.SMEM),
            pl.BlockSpec(memory_space=pltpu.SMEM),
            pl.BlockSpec((1, sub, _LANES), lambda b: (b, 0, 0)),
            pl.BlockSpec((1, sub, _LANES), lambda b: (b, 0, 0)),
        ],
        out_specs=pl.BlockSpec((1, sub, _LANES), lambda b: (b, 0, 0)),
        out_shape=jax.ShapeDtypeStruct((B, sub, _LANES), x_0.dtype),
    )(t, alpha_cumprod, x2, n2)
    return (out.reshape(x_0.shape), noise, t)


# speedup vs baseline: 1.1560x; 1.1560x over previous
import jax
import jax.numpy as jnp
from jax.experimental import pallas as pl
from jax.experimental.pallas import tpu as pltpu

_LANES = 128
_BS = 8


def _body(t_ref, alpha_ref, x_ref, n_ref, out_ref):
    b0 = pl.program_id(0) * _BS
    for r in range(_BS):
        a = alpha_ref[t_ref[b0 + r]]
        sa = jnp.sqrt(a)
        sn = jnp.sqrt(1.0 - a)
        out_ref[r] = sa * x_ref[r] + sn * n_ref[r]


def kernel(x_0, noise, t, alpha_cumprod):
    B = x_0.shape[0]
    per = x_0.size // B
    sub = per // _LANES
    x2 = x_0.reshape(B, sub, _LANES)
    n2 = noise.reshape(B, sub, _LANES)
    out = pl.pallas_call(
        _body,
        grid=(B // _BS,),
        in_specs=[
            pl.BlockSpec(memory_space=pltpu.SMEM),
            pl.BlockSpec(memory_space=pltpu.SMEM),
            pl.BlockSpec((_BS, sub, _LANES), lambda b: (b, 0, 0)),
            pl.BlockSpec((_BS, sub, _LANES), lambda b: (b, 0, 0)),
        ],
        out_specs=pl.BlockSpec((_BS, sub, _LANES), lambda b: (b, 0, 0)),
        out_shape=jax.ShapeDtypeStruct((B, sub, _LANES), x_0.dtype),
    )(t, alpha_cumprod, x2, n2)
    return (out.reshape(x_0.shape), noise, t)
